# Initial kernel scaffold; baseline (speedup 1.0000x reference)
import jax
import jax.numpy as jnp
from jax.experimental import pallas as pl


def kernel(x):
    raise NotImplementedError("write your pallas kernel here")


# speedup vs baseline: 5.1228x; 5.1228x over previous
import functools

import jax
import jax.numpy as jnp
from jax.experimental import pallas as pl
from jax.experimental.pallas import tpu as pltpu

_K = 64
_ROWS_PER_BLOCK = 8


def _topk_mask_body(x_ref, o_ref):
    x = x_ref[...]
    r = jnp.maximum(x, 0.0)
    xi = jax.lax.bitcast_convert_type(r, jnp.int32)

    t = jnp.zeros((x.shape[0], 1), dtype=jnp.int32)
    for b in range(30, -1, -1):
        cand = t | (1 << b)
        cnt = jnp.sum((xi >= cand).astype(jnp.int32), axis=1, keepdims=True)
        t = jnp.where(cnt >= _K, cand, t)

    o_ref[...] = jnp.where(xi >= t, r, 0.0)


@jax.jit
def kernel(x):
    n_rows, n_cols = x.shape
    grid = (n_rows // _ROWS_PER_BLOCK,)
    return pl.pallas_call(
        _topk_mask_body,
        grid=grid,
        in_specs=[pl.BlockSpec((_ROWS_PER_BLOCK, n_cols), lambda i: (i, 0))],
        out_specs=pl.BlockSpec((_ROWS_PER_BLOCK, n_cols), lambda i: (i, 0)),
        out_shape=jax.ShapeDtypeStruct((n_rows, n_cols), x.dtype),
        compiler_params=pltpu.CompilerParams(
            dimension_semantics=("arbitrary",),
        ),
    )(x)


# speedup vs baseline: 7.9630x; 1.5544x over previous
import jax
import jax.numpy as jnp
from jax import lax
from jax.experimental import pallas as pl
from jax.experimental.pallas import tpu as pltpu
from jax.experimental.pallas import tpu_sc as plsc

_K = 64
_L = 16
_NC = 2
_NS = 16
_NW = _NC * _NS
_CAP = 256
_NSEG = 8


def _sc_body(x_hbm, out_hbm, row_v, obuf_v, grp_v, cidx_v):
    n_rows, n_cols = x_hbm.shape
    n_vregs = n_cols // _L
    rows_per_w = n_rows // _NW
    seg_vregs = n_vregs // _NSEG

    wid = lax.axis_index("s") * _NC + lax.axis_index("c")
    iota = lax.iota(jnp.int32, _L)
    zero_f = jnp.zeros((_L,), jnp.float32)
    zero_i = jnp.zeros((_L,), jnp.int32)

    @plsc.parallel_loop(0, n_vregs)
    def _(i):
        obuf_v[pl.ds(i * _L, _L)] = zero_f

    def do_row(r, carry):
        row = wid * rows_per_w + r
        pltpu.sync_copy(x_hbm.at[row], row_v)

        for seg in range(_NSEG):
            @plsc.parallel_loop(0, seg_vregs, carry=zero_f)
            def acc(i, a):
                v = row_v[pl.ds((seg * seg_vregs + i) * _L, _L)]
                return jnp.maximum(a, v)
            grp_v[pl.ds(seg * _L, _L)] = acc

        gi = [
            lax.bitcast_convert_type(grp_v[pl.ds(k * _L, _L)], jnp.int32)
            for k in range(_NSEG)
        ]
        t = zero_i
        for b in range(30, 12, -1):
            cand = t | (1 << b)
            cnt = zero_i
            for k in range(_NSEG):
                cnt = cnt + plsc.all_reduce_population_count(gi[k] >= cand)
            t = jnp.where(cnt >= _K, cand, t)
        mstar = lax.bitcast_convert_type(t, jnp.float32)

        for c in range(_CAP // _L):
            cidx_v[pl.ds(c * _L, _L)] = zero_i

        @plsc.parallel_loop(0, n_vregs, carry=zero_i)
        def cnt_splat(j, cnt):
            v = row_v[pl.ds(j * _L, _L)]
            m = v >= mstar
            pos = cnt + plsc.cumsum(m.astype(jnp.int32)) - 1
            pos = jnp.minimum(jnp.maximum(pos, 0), _CAP - 1)
            plsc.store_scatter(cidx_v, [pos], iota + j * _L, mask=m)
            return cnt + plsc.all_reduce_population_count(m)

        ci = []
        for c in range(_CAP // _L):
            iv = cidx_v[pl.ds(c * _L, _L)]
            vals = plsc.load_gather(row_v, [iv])
            valid = (iota + c * _L) < cnt_splat
            ci.append(lax.bitcast_convert_type(jnp.where(valid, vals, 0.0), jnp.int32))

        t = zero_i
        for b in range(30, -1, -1):
            cand = t | (1 << b)
            cnt = zero_i
            for c in range(_CAP // _L):
                cnt = cnt + plsc.all_reduce_population_count(ci[c] >= cand)
            t = jnp.where(cnt >= _K, cand, t)

        for c in range(_CAP // _L):
            iv = cidx_v[pl.ds(c * _L, _L)]
            keep = ci[c] >= t
            plsc.store_scatter(
                obuf_v, [iv], lax.bitcast_convert_type(ci[c], jnp.float32), mask=keep
            )
        pltpu.sync_copy(obuf_v, out_hbm.at[row])
        for c in range(_CAP // _L):
            iv = cidx_v[pl.ds(c * _L, _L)]
            plsc.store_scatter(obuf_v, [iv], zero_f)
        return carry

    lax.fori_loop(0, rows_per_w, do_row, jnp.int32(0))


@jax.jit
def kernel(x):
    n_rows, n_cols = x.shape
    f = pl.kernel(
        _sc_body,
        out_type=jax.ShapeDtypeStruct((n_rows, n_cols), x.dtype),
        mesh=plsc.VectorSubcoreMesh(
            core_axis_name="c", subcore_axis_name="s",
            num_cores=_NC, num_subcores=_NS,
        ),
        compiler_params=pltpu.CompilerParams(needs_layout_passes=False),
        scratch_types=[
            pltpu.VMEM((n_cols,), jnp.float32),
            pltpu.VMEM((n_cols,), jnp.float32),
            pltpu.VMEM((_NSEG * _L,), jnp.float32),
            pltpu.VMEM((_CAP,), jnp.int32),
        ],
    )
    return f(x)


# speedup vs baseline: 14.2367x; 1.7879x over previous
import jax
import jax.numpy as jnp
from jax import lax
from jax.experimental import pallas as pl
from jax.experimental.pallas import tpu as pltpu
from jax.experimental.pallas import tpu_sc as plsc

_K = 64
_L = 16
_NC = 2
_NS = 16
_NW = _NC * _NS
_CAP = 256
_NSEG = 8


def _sc_body(x_hbm, out_hbm, row_v, obuf_v, grp_v, cidx_v):
    n_rows, n_cols = x_hbm.shape
    n_vregs = n_cols // _L
    rows_per_w = n_rows // _NW
    seg_vregs = n_vregs // _NSEG

    wid = lax.axis_index("s") * _NC + lax.axis_index("c")
    iota = lax.iota(jnp.int32, _L)
    zero_f = jnp.zeros((_L,), jnp.float32)
    zero_i = jnp.zeros((_L,), jnp.int32)

    @plsc.parallel_loop(0, n_vregs, unroll=8)
    def _(i):
        obuf_v[pl.ds(i * _L, _L)] = zero_f

    def do_row(r, carry):
        row = wid * rows_per_w + r
        pltpu.sync_copy(x_hbm.at[row], row_v)

        for seg in range(_NSEG):
            @plsc.parallel_loop(0, seg_vregs, unroll=8, carry=zero_f)
            def acc(i, a):
                v = row_v[pl.ds((seg * seg_vregs + i) * _L, _L)]
                return jnp.maximum(a, v)
            grp_v[pl.ds(seg * _L, _L)] = acc

        gi = [
            lax.bitcast_convert_type(grp_v[pl.ds(k * _L, _L)], jnp.int32)
            for k in range(_NSEG)
        ]
        t = zero_i
        for b in range(30, 12, -1):
            cand = t | (1 << b)
            cnt = zero_i
            for k in range(_NSEG):
                cnt = cnt + plsc.all_reduce_population_count(gi[k] >= cand)
            t = jnp.where(cnt >= _K, cand, t)
        mstar = lax.bitcast_convert_type(t, jnp.float32)

        for c in range(_CAP // _L):
            cidx_v[pl.ds(c * _L, _L)] = zero_i

        @plsc.parallel_loop(0, n_vregs, unroll=4, carry=zero_i)
        def cnt_splat(j, cnt):
            v = row_v[pl.ds(j * _L, _L)]
            m = v >= mstar
            pos = cnt + plsc.cumsum(m.astype(jnp.int32)) - 1
            pos = jnp.minimum(jnp.maximum(pos, 0), _CAP - 1)
            plsc.store_scatter(cidx_v, [pos], iota + j * _L, mask=m)
            return cnt + plsc.all_reduce_population_count(m)

        ci = []
        for c in range(_CAP // _L):
            iv = cidx_v[pl.ds(c * _L, _L)]
            vals = plsc.load_gather(row_v, [iv])
            valid = (iota + c * _L) < cnt_splat
            ci.append(lax.bitcast_convert_type(jnp.where(valid, vals, 0.0), jnp.int32))

        t = zero_i
        for b in range(30, -1, -1):
            cand = t | (1 << b)
            cnt = zero_i
            for c in range(_CAP // _L):
                cnt = cnt + plsc.all_reduce_population_count(ci[c] >= cand)
            t = jnp.where(cnt >= _K, cand, t)

        for c in range(_CAP // _L):
            iv = cidx_v[pl.ds(c * _L, _L)]
            keep = ci[c] >= t
            plsc.store_scatter(
                obuf_v, [iv], lax.bitcast_convert_type(ci[c], jnp.float32), mask=keep
            )
        pltpu.sync_copy(obuf_v, out_hbm.at[row])
        for c in range(_CAP // _L):
            iv = cidx_v[pl.ds(c * _L, _L)]
            plsc.store_scatter(obuf_v, [iv], zero_f)
        return carry

    lax.fori_loop(0, rows_per_w, do_row, jnp.int32(0))


@jax.jit
def kernel(x):
    n_rows, n_cols = x.shape
    f = pl.kernel(
        _sc_body,
        out_type=jax.ShapeDtypeStruct((n_rows, n_cols), x.dtype),
        mesh=plsc.VectorSubcoreMesh(
            core_axis_name="c", subcore_axis_name="s",
            num_cores=_NC, num_subcores=_NS,
        ),
        compiler_params=pltpu.CompilerParams(needs_layout_passes=False),
        scratch_types=[
            pltpu.VMEM((n_cols,), jnp.float32),
            pltpu.VMEM((n_cols,), jnp.float32),
            pltpu.VMEM((_NSEG * _L,), jnp.float32),
            pltpu.VMEM((_CAP,), jnp.int32),
        ],
    )
    return f(x)


# speedup vs baseline: 14.6331x; 1.0278x over previous
import jax
import jax.numpy as jnp
from jax import lax
from jax.experimental import pallas as pl
from jax.experimental.pallas import tpu as pltpu
from jax.experimental.pallas import tpu_sc as plsc

_K = 64
_L = 16
_NC = 2
_NS = 16
_NW = _NC * _NS
_CAP = 256
_NSEG = 8


def _sc_body(x_hbm, out_hbm, row_v, obuf_v, grp_v, cidx_v):
    n_rows, n_cols = x_hbm.shape
    n_vregs = n_cols // _L
    rows_per_w = n_rows // _NW
    seg_vregs = n_vregs // _NSEG

    wid = lax.axis_index("s") * _NC + lax.axis_index("c")
    iota = lax.iota(jnp.int32, _L)
    zero_f = jnp.zeros((_L,), jnp.float32)
    zero_i = jnp.zeros((_L,), jnp.int32)

    @plsc.parallel_loop(0, n_vregs, unroll=8)
    def _(i):
        obuf_v[pl.ds(i * _L, _L)] = zero_f

    def do_row(r, carry):
        row = wid * rows_per_w + r
        pltpu.sync_copy(x_hbm.at[row], row_v)

        for seg in range(_NSEG):
            @plsc.parallel_loop(0, seg_vregs, unroll=8, carry=zero_f)
            def acc(i, a):
                v = row_v[pl.ds((seg * seg_vregs + i) * _L, _L)]
                return jnp.maximum(a, v)
            grp_v[pl.ds(seg * _L, _L)] = acc

        gi = [
            lax.bitcast_convert_type(grp_v[pl.ds(k * _L, _L)], jnp.int32)
            for k in range(_NSEG)
        ]
        t = zero_i
        for b in range(30, 12, -1):
            cand = t | (1 << b)
            cnt = zero_i
            for k in range(_NSEG):
                cnt = cnt + plsc.all_reduce_population_count(gi[k] >= cand)
            t = jnp.where(cnt >= _K, cand, t)
        mstar = lax.bitcast_convert_type(t, jnp.float32)

        for c in range(_CAP // _L):
            cidx_v[pl.ds(c * _L, _L)] = zero_i

        @plsc.parallel_loop(0, n_vregs, unroll=8, carry=zero_i)
        def cnt_splat(j, cnt):
            v = row_v[pl.ds(j * _L, _L)]
            m = v >= mstar
            pos = cnt + plsc.cumsum(m.astype(jnp.int32)) - 1
            pos = jnp.minimum(jnp.maximum(pos, 0), _CAP - 1)
            plsc.store_scatter(cidx_v, [pos], iota + j * _L, mask=m)
            return cnt + plsc.all_reduce_population_count(m)

        ci = []
        for c in range(_CAP // _L):
            iv = cidx_v[pl.ds(c * _L, _L)]
            vals = plsc.load_gather(row_v, [iv])
            valid = (iota + c * _L) < cnt_splat
            ci.append(lax.bitcast_convert_type(jnp.where(valid, vals, 0.0), jnp.int32))

        t = zero_i
        for b in range(30, -1, -1):
            cand = t | (1 << b)
            cnt = zero_i
            for c in range(_CAP // _L):
                cnt = cnt + plsc.all_reduce_population_count(ci[c] >= cand)
            t = jnp.where(cnt >= _K, cand, t)

        for c in range(_CAP // _L):
            iv = cidx_v[pl.ds(c * _L, _L)]
            keep = ci[c] >= t
            plsc.store_scatter(
                obuf_v, [iv], lax.bitcast_convert_type(ci[c], jnp.float32), mask=keep
            )
        pltpu.sync_copy(obuf_v, out_hbm.at[row])
        for c in range(_CAP // _L):
            iv = cidx_v[pl.ds(c * _L, _L)]
            plsc.store_scatter(obuf_v, [iv], zero_f)
        return carry

    lax.fori_loop(0, rows_per_w, do_row, jnp.int32(0))


@jax.jit
def kernel(x):
    n_rows, n_cols = x.shape
    f = pl.kernel(
        _sc_body,
        out_type=jax.ShapeDtypeStruct((n_rows, n_cols), x.dtype),
        mesh=plsc.VectorSubcoreMesh(
            core_axis_name="c", subcore_axis_name="s",
            num_cores=_NC, num_subcores=_NS,
        ),
        compiler_params=pltpu.CompilerParams(needs_layout_passes=False),
        scratch_types=[
            pltpu.VMEM((n_cols,), jnp.float32),
            pltpu.VMEM((n_cols,), jnp.float32),
            pltpu.VMEM((_NSEG * _L,), jnp.float32),
            pltpu.VMEM((_CAP,), jnp.int32),
        ],
    )
    return f(x)


# speedup vs baseline: 18.1709x; 1.2418x over previous
import jax
import jax.numpy as jnp
from jax import lax
from jax.experimental import pallas as pl
from jax.experimental.pallas import tpu as pltpu
from jax.experimental.pallas import tpu_sc as plsc

_K = 64
_L = 16
_NC = 2
_NS = 16
_NW = _NC * _NS
_CAP = 256
_NSEG = 8


def _sc_body(x_hbm, out_hbm, row_v, obuf_v, grp_v, cidx_v, pidx_v,
             sem_in, sem_out):
    n_rows, n_cols = x_hbm.shape
    n_vregs = n_cols // _L
    rows_per_w = n_rows // _NW
    seg_vregs = n_vregs // _NSEG
    ncand = _CAP // _L

    wid = lax.axis_index("s") * _NC + lax.axis_index("c")
    row0 = wid * rows_per_w
    iota = lax.iota(jnp.int32, _L)
    zero_f = jnp.zeros((_L,), jnp.float32)
    zero_i = jnp.zeros((_L,), jnp.int32)

    pltpu.make_async_copy(
        x_hbm.at[row0], row_v.at[pl.ds(0, n_cols)], sem_in
    ).start()

    @plsc.parallel_loop(0, n_vregs, unroll=8)
    def _(i):
        obuf_v[pl.ds(i * _L, _L)] = zero_f

    def do_row(r, carry):
        row = row0 + r
        base = (r % 2) * n_cols
        nbase = ((r + 1) % 2) * n_cols
        pltpu.make_async_copy(
            x_hbm.at[row], row_v.at[pl.ds(base, n_cols)], sem_in
        ).wait()

        @pl.when(r < rows_per_w - 1)
        def _():
            pltpu.make_async_copy(
                x_hbm.at[row + 1], row_v.at[pl.ds(nbase, n_cols)], sem_in
            ).start()

        for seg in range(_NSEG):
            @plsc.parallel_loop(0, seg_vregs, unroll=8, carry=zero_f)
            def acc(i, a):
                v = row_v[pl.ds(base + (seg * seg_vregs + i) * _L, _L)]
                return jnp.maximum(a, v)
            grp_v[pl.ds(seg * _L, _L)] = acc

        gi = [
            lax.bitcast_convert_type(grp_v[pl.ds(k * _L, _L)], jnp.int32)
            for k in range(_NSEG)
        ]
        t = zero_i
        for b in range(30, 12, -1):
            cand = t | (1 << b)
            cnt = zero_i
            for k in range(_NSEG):
                cnt = cnt + plsc.all_reduce_population_count(gi[k] >= cand)
            t = jnp.where(cnt >= _K, cand, t)
        mstar = lax.bitcast_convert_type(t, jnp.float32)

        for c in range(ncand):
            cidx_v[pl.ds(c * _L, _L)] = zero_i

        @plsc.parallel_loop(0, n_vregs, unroll=8, carry=zero_i)
        def cnt_splat(j, cnt):
            v = row_v[pl.ds(base + j * _L, _L)]
            m = v >= mstar
            pos = cnt + plsc.cumsum(m.astype(jnp.int32)) - 1
            pos = jnp.minimum(jnp.maximum(pos, 0), _CAP - 1)
            plsc.store_scatter(cidx_v, [pos], iota + j * _L, mask=m)
            return cnt + plsc.all_reduce_population_count(m)

        base_splat = zero_i + base
        ci = []
        for c in range(ncand):
            iv = cidx_v[pl.ds(c * _L, _L)]
            vals = plsc.load_gather(row_v, [iv + base_splat])
            valid = (iota + c * _L) < cnt_splat
            ci.append(
                lax.bitcast_convert_type(jnp.where(valid, vals, 0.0), jnp.int32)
            )

        t = zero_i
        for b in range(30, -1, -1):
            cand = t | (1 << b)
            cnt = zero_i
            for c in range(ncand):
                cnt = cnt + plsc.all_reduce_population_count(ci[c] >= cand)
            t = jnp.where(cnt >= _K, cand, t)

        @pl.when(r > 0)
        def _():
            pltpu.make_async_copy(obuf_v, out_hbm.at[row - 1], sem_out).wait()
            for c in range(ncand):
                iv = pidx_v[pl.ds(c * _L, _L)]
                plsc.store_scatter(obuf_v, [iv], zero_f)

        for c in range(ncand):
            iv = cidx_v[pl.ds(c * _L, _L)]
            keep = ci[c] >= t
            plsc.store_scatter(
                obuf_v, [iv], lax.bitcast_convert_type(ci[c], jnp.float32),
                mask=keep,
            )
        pltpu.make_async_copy(obuf_v, out_hbm.at[row], sem_out).start()
        for c in range(ncand):
            pidx_v[pl.ds(c * _L, _L)] = cidx_v[pl.ds(c * _L, _L)]
        return carry

    lax.fori_loop(0, rows_per_w, do_row, jnp.int32(0))
    pltpu.make_async_copy(
        obuf_v, out_hbm.at[row0 + rows_per_w - 1], sem_out
    ).wait()


@jax.jit
def kernel(x):
    n_rows, n_cols = x.shape
    f = pl.kernel(
        _sc_body,
        out_type=jax.ShapeDtypeStruct((n_rows, n_cols), x.dtype),
        mesh=plsc.VectorSubcoreMesh(
            core_axis_name="c", subcore_axis_name="s",
            num_cores=_NC, num_subcores=_NS,
        ),
        compiler_params=pltpu.CompilerParams(needs_layout_passes=False),
        scratch_types=[
            pltpu.VMEM((2 * n_cols,), jnp.float32),
            pltpu.VMEM((n_cols,), jnp.float32),
            pltpu.VMEM((_NSEG * _L,), jnp.float32),
            pltpu.VMEM((_CAP,), jnp.int32),
            pltpu.VMEM((_CAP,), jnp.int32),
            pltpu.SemaphoreType.DMA,
            pltpu.SemaphoreType.DMA,
        ],
    )
    return f(x)


# speedup vs baseline: 19.3725x; 1.0661x over previous
import jax
import jax.numpy as jnp
from jax import lax
from jax.experimental import pallas as pl
from jax.experimental.pallas import tpu as pltpu
from jax.experimental.pallas import tpu_sc as plsc

_K = 64
_L = 16
_NC = 2
_NS = 16
_NW = _NC * _NS
_SLOT = 32
_CAP = _SLOT * _L
_CCAP = 256
_NSEG = 8


def _sc_body(x_hbm, out_hbm, row_v, obuf_v, grp_v, cidx_v, cidx2_v, cval2_v,
             pidx_v, sem_in, sem_out):
    n_rows, n_cols = x_hbm.shape
    n_vregs = n_cols // _L
    rows_per_w = n_rows // _NW
    seg_vregs = n_vregs // _NSEG
    ncand = _CCAP // _L

    wid = lax.axis_index("s") * _NC + lax.axis_index("c")
    row0 = wid * rows_per_w
    iota = lax.iota(jnp.int32, _L)
    zero_f = jnp.zeros((_L,), jnp.float32)
    zero_i = jnp.zeros((_L,), jnp.int32)

    pltpu.make_async_copy(
        x_hbm.at[row0], row_v.at[pl.ds(0, n_cols)], sem_in
    ).start()

    @plsc.parallel_loop(0, n_vregs, unroll=8)
    def _(i):
        obuf_v[pl.ds(i * _L, _L)] = zero_f

    def do_row(r, carry):
        row = row0 + r
        base = (r % 2) * n_cols
        nbase = ((r + 1) % 2) * n_cols
        pltpu.make_async_copy(
            x_hbm.at[row], row_v.at[pl.ds(base, n_cols)], sem_in
        ).wait()

        @pl.when(r < rows_per_w - 1)
        def _():
            pltpu.make_async_copy(
                x_hbm.at[row + 1], row_v.at[pl.ds(nbase, n_cols)], sem_in
            ).start()

        for seg in range(_NSEG):
            @plsc.parallel_loop(0, seg_vregs, unroll=8, carry=zero_f)
            def acc(i, a):
                v = row_v[pl.ds(base + (seg * seg_vregs + i) * _L, _L)]
                return jnp.maximum(a, v)
            grp_v[pl.ds(seg * _L, _L)] = acc

        gi = [
            lax.bitcast_convert_type(grp_v[pl.ds(k * _L, _L)], jnp.int32)
            for k in range(_NSEG)
        ]
        t = zero_i
        for b in range(30, 12, -1):
            cand = t | (1 << b)
            cnt = zero_i
            for k in range(_NSEG):
                cnt = cnt + plsc.all_reduce_population_count(gi[k] >= cand)
            t = jnp.where(cnt >= _K, cand, t)
        mstar = lax.bitcast_convert_type(t, jnp.float32)

        for c in range(_CAP // _L):
            cidx_v[pl.ds(c * _L, _L)] = zero_i

        @plsc.parallel_loop(0, n_vregs, unroll=8, carry=zero_i)
        def cnt_lane(j, cl):
            v = row_v[pl.ds(base + j * _L, _L)]
            m = v >= mstar
            pos = jnp.minimum(cl, _SLOT - 1) * _L + iota
            plsc.store_scatter(cidx_v, [pos], iota + j * _L, mask=m)
            return cl + m.astype(jnp.int32)

        for c in range(ncand):
            cidx2_v[pl.ds(c * _L, _L)] = zero_i
            cval2_v[pl.ds(c * _L, _L)] = zero_f

        base_splat = zero_i + base

        @plsc.parallel_loop(0, _SLOT, unroll=4, carry=zero_i)
        def _(s, cnt):
            iv = cidx_v[pl.ds(s * _L, _L)]
            valid = cnt_lane > s
            vals = plsc.load_gather(row_v, [iv + base_splat])
            pos = cnt + plsc.cumsum(valid.astype(jnp.int32)) - 1
            pos = jnp.minimum(jnp.maximum(pos, 0), _CCAP - 1)
            plsc.store_scatter(cidx2_v, [pos], iv, mask=valid)
            plsc.store_scatter(cval2_v, [pos], vals, mask=valid)
            return cnt + plsc.all_reduce_population_count(valid)

        ci = [
            lax.bitcast_convert_type(cval2_v[pl.ds(c * _L, _L)], jnp.int32)
            for c in range(ncand)
        ]
        t = zero_i
        for b in range(30, -1, -1):
            cand = t | (1 << b)
            cnt = zero_i
            for c in range(ncand):
                cnt = cnt + plsc.all_reduce_population_count(ci[c] >= cand)
            t = jnp.where(cnt >= _K, cand, t)

        @pl.when(r > 0)
        def _():
            pltpu.make_async_copy(obuf_v, out_hbm.at[row - 1], sem_out).wait()
            for c in range(ncand):
                iv = pidx_v[pl.ds(c * _L, _L)]
                plsc.store_scatter(obuf_v, [iv], zero_f)

        for c in range(ncand):
            iv = cidx2_v[pl.ds(c * _L, _L)]
            keep = ci[c] >= t
            plsc.store_scatter(
                obuf_v, [iv], lax.bitcast_convert_type(ci[c], jnp.float32),
                mask=keep,
            )
        pltpu.make_async_copy(obuf_v, out_hbm.at[row], sem_out).start()
        for c in range(ncand):
            pidx_v[pl.ds(c * _L, _L)] = cidx2_v[pl.ds(c * _L, _L)]
        return carry

    lax.fori_loop(0, rows_per_w, do_row, jnp.int32(0))
    pltpu.make_async_copy(
        obuf_v, out_hbm.at[row0 + rows_per_w - 1], sem_out
    ).wait()


@jax.jit
def kernel(x):
    n_rows, n_cols = x.shape
    f = pl.kernel(
        _sc_body,
        out_type=jax.ShapeDtypeStruct((n_rows, n_cols), x.dtype),
        mesh=plsc.VectorSubcoreMesh(
            core_axis_name="c", subcore_axis_name="s",
            num_cores=_NC, num_subcores=_NS,
        ),
        compiler_params=pltpu.CompilerParams(needs_layout_passes=False),
        scratch_types=[
            pltpu.VMEM((2 * n_cols,), jnp.float32),
            pltpu.VMEM((n_cols,), jnp.float32),
            pltpu.VMEM((_NSEG * _L,), jnp.float32),
            pltpu.VMEM((_CAP,), jnp.int32),
            pltpu.VMEM((_CCAP,), jnp.int32),
            pltpu.VMEM((_CCAP,), jnp.float32),
            pltpu.VMEM((_CCAP,), jnp.int32),
            pltpu.SemaphoreType.DMA,
            pltpu.SemaphoreType.DMA,
        ],
    )
    return f(x)
